# baseline (speedup 1.0000x reference)
import functools

import jax
import jax.numpy as jnp
from jax import lax
from jax.experimental import pallas as pl
from jax.experimental.pallas import tpu as pltpu
from jax.experimental.pallas import tpu_sc as plsc

B = 1024
L = 200
D = 64
NC = 2
NS = 16
NW = NC * NS
ROWS_PER_W = B // NW
LANES = 16


def kernel(input, token_table, pos_table):
    idx_flat = input.reshape(B * L)
    mesh = plsc.VectorSubcoreMesh(core_axis_name="c", subcore_axis_name="s")

    @functools.partial(
        pl.kernel,
        out_type=jax.ShapeDtypeStruct((B * L, D), jnp.float32),
        mesh=mesh,
        compiler_params=pltpu.CompilerParams(use_tc_tiling_on_sc=False),
        scratch_types=[
            pltpu.VMEM((L * ROWS_PER_W,), jnp.int32),
            pltpu.VMEM((L, D), jnp.float32),
            pltpu.VMEM((L, D), jnp.float32),
            pltpu.SemaphoreType.DMA,
        ],
    )
    def emb_kernel(idx_hbm, tok_hbm, pos_hbm, out_hbm, idx_v, pos_v, rows_v, sem):
        wid = lax.axis_index("s") * NC + lax.axis_index("c")
        base = wid * (L * ROWS_PER_W)
        pltpu.sync_copy(idx_hbm.at[pl.ds(base, L * ROWS_PER_W)], idx_v)
        pltpu.sync_copy(pos_hbm, pos_v)

        @pl.loop(0, ROWS_PER_W)
        def _row(r):
            off = r * L
            c1 = pltpu.async_copy(
                tok_hbm.at[idx_v.at[pl.ds(off, 128)]],
                rows_v.at[pl.ds(0, 128)], sem)
            c2 = pltpu.async_copy(
                tok_hbm.at[idx_v.at[pl.ds(off + 128, L - 128)]],
                rows_v.at[pl.ds(128, L - 128)], sem)
            c1.wait()
            c2.wait()

            @pl.loop(0, L)
            def _add_row(i):
                @pl.loop(0, D, step=LANES)
                def _add_chunk(j):
                    slc = (pl.ds(i, 1), pl.ds(j, LANES))
                    rows_v.at[*slc][...] = (
                        rows_v.at[*slc][...] + pos_v.at[*slc][...])

            pltpu.sync_copy(rows_v, out_hbm.at[pl.ds(base + off, L)])

    out = emb_kernel(idx_flat, token_table, pos_table)
    return out.reshape(B, L, D)
